# baseline (speedup 1.0000x reference)
import functools

import jax
import jax.numpy as jnp
from jax import lax
from jax.experimental import pallas as pl
from jax.experimental.pallas import tpu as pltpu
from jax.experimental.pallas import tpu_sc as plsc

B = 16384
D = 128
NC = 2
NS = 16
NW = NC * NS
PER_W = B // NW
C = 128
NCHUNK = PER_W // C
L = 16

_mesh = plsc.VectorSubcoreMesh(core_axis_name="c", subcore_axis_name="s")


@functools.partial(
    pl.kernel,
    mesh=_mesh,
    out_type=jax.ShapeDtypeStruct((B,), jnp.float32),
    compiler_params=pltpu.CompilerParams(needs_layout_passes=False),
    scratch_types=[
        pltpu.VMEM((PER_W,), jnp.int32),
        pltpu.VMEM((PER_W,), jnp.int32),
        pltpu.VMEM((2, C, D), jnp.float32),
        pltpu.VMEM((2, C, D), jnp.float32),
        pltpu.VMEM((PER_W,), jnp.float32),
        pltpu.VMEM((L * L,), jnp.float32),
        pltpu.SemaphoreType.DMA,
        pltpu.SemaphoreType.DMA,
        pltpu.SemaphoreType.DMA,
        pltpu.SemaphoreType.DMA,
        pltpu.SemaphoreType.DMA,
        pltpu.SemaphoreType.DMA,
        pltpu.SemaphoreType.DMA,
        pltpu.SemaphoreType.DMA,
    ],
)
def _mf_sc(uid_hbm, iid_hbm, utab_hbm, itab_hbm, out_hbm,
           idx_u, idx_i, rows_u, rows_i, out_v, tbuf,
           su0, su1, su2, su3, si0, si1, si2, si3):
    wid = lax.axis_index("s") * NC + lax.axis_index("c")
    base = wid * PER_W
    colbase = lax.iota(jnp.int32, L) * L
    sems_u = (su0, su1, su2, su3)
    sems_i = (si0, si1, si2, si3)

    pltpu.sync_copy(uid_hbm.at[pl.ds(base, PER_W)], idx_u)
    pltpu.sync_copy(iid_hbm.at[pl.ds(base, PER_W)], idx_i)

    subs = [
        (0, 32, 0, 0), (32, 32, 0, 32), (64, 32, 0, 64), (96, 32, 0, 96),
        (128, 128, 1, 0), (256, 128, 0, 0), (384, 96, 1, 0), (480, 32, 1, 96),
    ]
    descs = {}

    def fire(k):
        off, sz, buf, boff = subs[k]
        s = k % 4
        descs[k] = (
            pltpu.async_copy(utab_hbm.at[idx_u.at[pl.ds(off, sz)]],
                             rows_u.at[buf, pl.ds(boff, sz)], sems_u[s]),
            pltpu.async_copy(itab_hbm.at[idx_i.at[pl.ds(off, sz)]],
                             rows_i.at[buf, pl.ds(boff, sz)], sems_i[s]),
        )

    def wait(k):
        du, di = descs.pop(k)
        du.wait()
        di.wait()

    def compute(k):
        off, sz, buf, boff = subs[k]
        ru = rows_u.at[buf]
        ri = rows_i.at[buf]

        def _group(g, _, off=off, boff=boff, ru=ru, ri=ri):
            for l in range(L):
                r = boff + g * L + l
                acc = ru[r, pl.ds(0, L)] * ri[r, pl.ds(0, L)]
                for j in range(1, D // L):
                    acc = acc + ru[r, pl.ds(j * L, L)] * ri[r, pl.ds(j * L, L)]
                tbuf[pl.ds(l * L, L)] = acc
            out_vec = plsc.load_gather(tbuf, [colbase])
            for l in range(1, L):
                out_vec = out_vec + plsc.load_gather(tbuf, [colbase + l])
            out_v[pl.ds(off + g * L, L)] = 1.0 / (1.0 + jnp.exp(-out_vec))
            return 0

        lax.fori_loop(0, sz // L, _group, 0)

    fire(0), fire(1), fire(2), fire(3)
    wait(0); fire(4); compute(0)
    wait(1); compute(1)
    wait(2); compute(2)
    wait(3); compute(3); fire(5)
    wait(4); compute(4); fire(6); fire(7)
    wait(5); compute(5)
    wait(6); compute(6)
    wait(7); compute(7)

    pltpu.sync_copy(out_v, out_hbm.at[pl.ds(base, PER_W)])


def kernel(user_ids, item_ids, user_table, item_table):
    return _mf_sc(user_ids, item_ids, user_table, item_table)


# speedup vs baseline: 1.0433x; 1.0433x over previous
import functools

import jax
import jax.numpy as jnp
from jax import lax
from jax.experimental import pallas as pl
from jax.experimental.pallas import tpu as pltpu
from jax.experimental.pallas import tpu_sc as plsc

B = 16384
D = 128
NC = 2
NS = 16
NW = NC * NS
PER_W = B // NW
C = 128
NCHUNK = PER_W // C
L = 16

_mesh = plsc.VectorSubcoreMesh(core_axis_name="c", subcore_axis_name="s")


@functools.partial(
    pl.kernel,
    mesh=_mesh,
    out_type=jax.ShapeDtypeStruct((B,), jnp.float32),
    compiler_params=pltpu.CompilerParams(needs_layout_passes=False),
    scratch_types=[
        pltpu.VMEM((PER_W,), jnp.int32),
        pltpu.VMEM((PER_W,), jnp.int32),
        pltpu.VMEM((2, C, D), jnp.float32),
        pltpu.VMEM((2, C, D), jnp.float32),
        pltpu.VMEM((PER_W,), jnp.float32),
        pltpu.VMEM((L * L,), jnp.float32),
        pltpu.SemaphoreType.DMA,
        pltpu.SemaphoreType.DMA,
        pltpu.SemaphoreType.DMA,
        pltpu.SemaphoreType.DMA,
    ],
)
def _mf_sc(uid_hbm, iid_hbm, utab_hbm, itab_hbm, out_hbm,
           idx_u, idx_i, rows_u, rows_i, out_v, tbuf,
           sem_u0, sem_u1, sem_i0, sem_i1):
    wid = lax.axis_index("s") * NC + lax.axis_index("c")
    base = wid * PER_W
    colbase = lax.iota(jnp.int32, L) * L
    sems_u = (sem_u0, sem_u1)
    sems_i = (sem_i0, sem_i1)

    pltpu.sync_copy(uid_hbm.at[pl.ds(base, PER_W)], idx_u)
    pltpu.sync_copy(iid_hbm.at[pl.ds(base, PER_W)], idx_i)

    def fire(chunk):
        b = chunk % 2
        return (
            pltpu.async_copy(utab_hbm.at[idx_u.at[pl.ds(chunk * C, C)]],
                             rows_u.at[b], sems_u[b]),
            pltpu.async_copy(itab_hbm.at[idx_i.at[pl.ds(chunk * C, C)]],
                             rows_i.at[b], sems_i[b]),
        )

    pending = fire(0)
    for chunk in range(NCHUNK):
        b = chunk % 2
        du, di = pending
        du.wait()
        di.wait()
        if chunk + 1 < NCHUNK:
            pending = fire(chunk + 1)
        ru = rows_u.at[b]
        ri = rows_i.at[b]

        def _group(g, _, chunk=chunk, ru=ru, ri=ri):
            for l in range(L):
                r = g * L + l
                m = [ru[r, pl.ds(j * L, L)] * ri[r, pl.ds(j * L, L)]
                     for j in range(D // L)]
                while len(m) > 1:
                    m = [m[i] + m[i + 1] for i in range(0, len(m), 2)]
                tbuf[pl.ds(l * L, L)] = m[0]
            out_vec = plsc.load_gather(tbuf, [colbase])
            for l in range(1, L):
                out_vec = out_vec + plsc.load_gather(tbuf, [colbase + l])
            out_v[pl.ds(chunk * C + g * L, L)] = 1.0 / (1.0 + jnp.exp(-out_vec))
            return 0

        lax.fori_loop(0, C // L, _group, 0)

    pltpu.sync_copy(out_v, out_hbm.at[pl.ds(base, PER_W)])


def kernel(user_ids, item_ids, user_table, item_table):
    return _mf_sc(user_ids, item_ids, user_table, item_table)


# speedup vs baseline: 1.1013x; 1.0555x over previous
import functools

import jax
import jax.numpy as jnp
from jax import lax
from jax.experimental import pallas as pl
from jax.experimental.pallas import tpu as pltpu
from jax.experimental.pallas import tpu_sc as plsc

B = 16384
D = 128
NC = 2
NS = 16
NW = NC * NS
PER_W = B // NW
C = 128
NCHUNK = PER_W // C
L = 16

_mesh = plsc.VectorSubcoreMesh(core_axis_name="c", subcore_axis_name="s")


@functools.partial(
    pl.kernel,
    mesh=_mesh,
    out_type=jax.ShapeDtypeStruct((B,), jnp.float32),
    compiler_params=pltpu.CompilerParams(needs_layout_passes=False),
    scratch_types=[
        pltpu.VMEM((PER_W,), jnp.int32),
        pltpu.VMEM((PER_W,), jnp.int32),
        pltpu.VMEM((2, C, D), jnp.float32),
        pltpu.VMEM((2, C, D), jnp.float32),
        pltpu.VMEM((PER_W,), jnp.float32),
        pltpu.VMEM((L * L,), jnp.float32),
        pltpu.SemaphoreType.DMA,
        pltpu.SemaphoreType.DMA,
        pltpu.SemaphoreType.DMA,
        pltpu.SemaphoreType.DMA,
    ],
)
def _mf_sc(uid_hbm, iid_hbm, utab_hbm, itab_hbm, out_hbm,
           idx_u, idx_i, rows_u, rows_i, out_v, tbuf,
           sem_u0, sem_u1, sem_i0, sem_i1):
    wid = lax.axis_index("s") * NC + lax.axis_index("c")
    base = wid * PER_W
    colbase = lax.iota(jnp.int32, L) * L
    sems_u = (sem_u0, sem_u1)
    sems_i = (sem_i0, sem_i1)

    pltpu.sync_copy(uid_hbm.at[pl.ds(base, PER_W)], idx_u)
    pltpu.sync_copy(iid_hbm.at[pl.ds(base, PER_W)], idx_i)

    def fire(chunk):
        b = chunk % 2
        return (
            pltpu.async_copy(utab_hbm.at[idx_u.at[pl.ds(chunk * C, C)]],
                             rows_u.at[b], sems_u[b]),
            pltpu.async_copy(itab_hbm.at[idx_i.at[pl.ds(chunk * C, C)]],
                             rows_i.at[b], sems_i[b]),
        )

    pending = fire(0)
    for chunk in range(NCHUNK):
        b = chunk % 2
        du, di = pending
        du.wait()
        di.wait()
        if chunk + 1 < NCHUNK:
            pending = fire(chunk + 1)
        ru = rows_u.at[b]
        ri = rows_i.at[b]

        def _group(g, _, chunk=chunk, ru=ru, ri=ri):
            for l in range(L):
                r = g * L + l
                acc = ru[r, pl.ds(0, L)] * ri[r, pl.ds(0, L)]
                for j in range(1, D // L):
                    acc = acc + ru[r, pl.ds(j * L, L)] * ri[r, pl.ds(j * L, L)]
                tbuf[pl.ds(l * L, L)] = acc
            out_vec = plsc.load_gather(tbuf, [colbase])
            for l in range(1, L):
                out_vec = out_vec + plsc.load_gather(tbuf, [colbase + l])
            out_v[pl.ds(chunk * C + g * L, L)] = 1.0 / (1.0 + jnp.exp(-out_vec))
            return 0

        lax.fori_loop(0, C // L, _group, 0)

    pltpu.sync_copy(out_v, out_hbm.at[pl.ds(base, PER_W)])


def kernel(user_ids, item_ids, user_table, item_table):
    return _mf_sc(user_ids, item_ids, user_table, item_table)
